# baseline (device time: 73742 ns/iter reference)
import jax
import jax.numpy as jnp
from jax import lax
from jax.experimental import pallas as pl
from jax.experimental.pallas import tpu as pltpu

N_DEV = 4
SCALE = 0.08838834764831843


def _body(x_ref, wq_ref, wo_ref, k_ref, v_ref, out_ref,
          comm_ref, send_sems, recv_sems):
    my = lax.axis_index("i")
    left = lax.rem(my + N_DEV - 1, N_DEV)
    right = lax.rem(my + 1, N_DEV)

    barrier_sem = pltpu.get_barrier_semaphore()
    for nbr in (left, right):
        pl.semaphore_signal(
            barrier_sem, inc=1,
            device_id=(nbr,), device_id_type=pl.DeviceIdType.MESH,
        )
    pl.semaphore_wait(barrier_sem, 2)

    q_all = jnp.dot(x_ref[...], wq_ref[...], preferred_element_type=jnp.float32)

    heads = []
    for h in range(8):
        g = h // 4
        q = q_all[:, h * 128:(h + 1) * 128]
        k = k_ref[g]
        s = lax.dot_general(
            q, k, (((1,), (1,)), ((), ())),
            preferred_element_type=jnp.float32,
        ) * SCALE
        m = jnp.max(s, axis=1, keepdims=True)
        p = jnp.exp(s - m)
        l = jnp.sum(p, axis=1, keepdims=True)
        o = jnp.dot(p, v_ref[g], preferred_element_type=jnp.float32) / l
        heads.append(o)
    attn = jnp.concatenate(heads, axis=1)

    partial = jnp.dot(attn, wo_ref[...], preferred_element_type=jnp.float32)
    comm_ref[0] = partial
    out_ref[...] = partial

    for hop in range(N_DEV - 1):
        rdma = pltpu.make_async_remote_copy(
            src_ref=comm_ref.at[hop],
            dst_ref=comm_ref.at[hop + 1],
            send_sem=send_sems.at[hop],
            recv_sem=recv_sems.at[hop],
            device_id=(right,),
            device_id_type=pl.DeviceIdType.MESH,
        )
        rdma.start()
        rdma.wait()
        out_ref[...] = out_ref[...] + comm_ref[hop + 1]


def kernel(x, Wq, Wo, K_ext, V_ext):
    i = lax.axis_index("i")
    x2d = x[0]
    k_loc = lax.dynamic_slice_in_dim(K_ext[0], 2 * i, 2, axis=1)
    v_loc = lax.dynamic_slice_in_dim(V_ext[0], 2 * i, 2, axis=1)
    k_loc = jnp.transpose(k_loc, (1, 0, 2))
    v_loc = jnp.transpose(v_loc, (1, 0, 2))

    out = pl.pallas_call(
        _body,
        out_shape=jax.ShapeDtypeStruct((256, 1024), jnp.float32),
        in_specs=[pl.BlockSpec(memory_space=pltpu.VMEM)] * 5,
        out_specs=pl.BlockSpec(memory_space=pltpu.VMEM),
        scratch_shapes=[
            pltpu.VMEM((N_DEV, 256, 1024), jnp.float32),
            pltpu.SemaphoreType.DMA((N_DEV - 1,)),
            pltpu.SemaphoreType.DMA((N_DEV - 1,)),
        ],
        compiler_params=pltpu.CompilerParams(collective_id=0),
    )(x2d, Wq, Wo, k_loc, v_loc)
    return out[None]


# device time: 46899 ns/iter; 1.5724x vs baseline; 1.5724x over previous
import jax
import jax.numpy as jnp
from jax import lax
from jax.experimental import pallas as pl
from jax.experimental.pallas import tpu as pltpu

N_DEV = 4
SCALE = 0.08838834764831843


def _body(x_ref, wq_ref, wo_ref, k_ref, v_ref, out_ref,
          s1_buf, s2_buf, recv_buf, send_sems, recv_sems):
    my = lax.axis_index("i")
    partner_y = my - 2 * lax.rem(my, 2) + 1
    partner_x = 3 - my

    barrier_sem = pltpu.get_barrier_semaphore()
    for nbr in (partner_y, partner_x):
        pl.semaphore_signal(
            barrier_sem, inc=1,
            device_id=(nbr,), device_id_type=pl.DeviceIdType.MESH,
        )
    pl.semaphore_wait(barrier_sem, 2)

    q_all = jnp.dot(x_ref[...], wq_ref[...],
                    preferred_element_type=jnp.float32) * SCALE

    heads = []
    for h in range(8):
        g = h // 4
        q = q_all[:, h * 128:(h + 1) * 128]
        s = lax.dot_general(
            q, k_ref[g], (((1,), (1,)), ((), ())),
            preferred_element_type=jnp.float32,
        )
        m = jnp.max(s, axis=1, keepdims=True)
        p = jnp.exp(s - m)
        l = jnp.sum(p, axis=1, keepdims=True)
        o = jnp.dot(p, v_ref[g], preferred_element_type=jnp.float32) / l
        heads.append(o)
    attn = jnp.concatenate(heads, axis=1)

    partial = jnp.dot(attn, wo_ref[...], preferred_element_type=jnp.float32)

    s1_buf[0] = partial[:, :512]
    s1_buf[1] = partial[:, 512:]
    s1 = []
    for idx, tgt in ((0, partner_y), (1, partner_x)):
        rdma = pltpu.make_async_remote_copy(
            src_ref=s1_buf.at[idx], dst_ref=recv_buf.at[idx],
            send_sem=send_sems.at[idx], recv_sem=recv_sems.at[idx],
            device_id=(tgt,), device_id_type=pl.DeviceIdType.MESH,
        )
        rdma.start()
        s1.append(rdma)
    for rdma in s1:
        rdma.wait()

    s2_buf[0] = s1_buf[0] + recv_buf[0]
    s2_buf[1] = s1_buf[1] + recv_buf[1]
    s2 = []
    for idx, tgt in ((0, partner_x), (1, partner_y)):
        rdma = pltpu.make_async_remote_copy(
            src_ref=s2_buf.at[idx], dst_ref=recv_buf.at[idx + 2],
            send_sem=send_sems.at[idx + 2], recv_sem=recv_sems.at[idx + 2],
            device_id=(tgt,), device_id_type=pl.DeviceIdType.MESH,
        )
        rdma.start()
        s2.append(rdma)
    for rdma in s2:
        rdma.wait()

    out_ref[:, :512] = s2_buf[0] + recv_buf[2]
    out_ref[:, 512:] = s2_buf[1] + recv_buf[3]


def kernel(x, Wq, Wo, K_ext, V_ext):
    i = lax.axis_index("i")
    x2d = x[0]
    k_loc = lax.dynamic_slice_in_dim(K_ext[0], 2 * i, 2, axis=1)
    v_loc = lax.dynamic_slice_in_dim(V_ext[0], 2 * i, 2, axis=1)
    k_loc = jnp.transpose(k_loc, (1, 0, 2))
    v_loc = jnp.transpose(v_loc, (1, 0, 2))

    out = pl.pallas_call(
        _body,
        out_shape=jax.ShapeDtypeStruct((256, 1024), jnp.float32),
        in_specs=[pl.BlockSpec(memory_space=pltpu.VMEM)] * 5,
        out_specs=pl.BlockSpec(memory_space=pltpu.VMEM),
        scratch_shapes=[
            pltpu.VMEM((2, 256, 512), jnp.float32),
            pltpu.VMEM((2, 256, 512), jnp.float32),
            pltpu.VMEM((4, 256, 512), jnp.float32),
            pltpu.SemaphoreType.DMA((4,)),
            pltpu.SemaphoreType.DMA((4,)),
        ],
        compiler_params=pltpu.CompilerParams(collective_id=0),
    )(x2d, Wq, Wo, k_loc, v_loc)
    return out[None]
